# baseline (speedup 1.0000x reference)
import functools

import jax
import jax.numpy as jnp
from jax import lax
from jax.experimental import pallas as pl
from jax.experimental.pallas import tpu as pltpu
from jax.experimental.pallas import tpu_sc as plsc

EMB = 64
REPR = 32
NREL = 64
NGRP = 8
RPG = NREL // NGRP
BLK = 1024
REGION = 57344
BPSC = REGION // BLK


def _sc_sort_gather(pairs, attr, node_emb):
    nw = pairs.shape[0]
    epw = pairs.shape[1] // 2
    info = plsc.get_sparse_core_info()
    nsc = info.num_cores
    tpsc = info.num_subcores
    tslice = REGION // tpsc
    n_ch = tslice // 128
    e_pad = nsc * REGION
    nvec = epw // 16
    mesh = plsc.VectorSubcoreMesh(core_axis_name="c", subcore_axis_name="s")
    cap = epw + 16

    @functools.partial(
        pl.kernel,
        out_type=(
            jax.ShapeDtypeStruct((e_pad, EMB), jnp.float32),
            jax.ShapeDtypeStruct((e_pad, EMB), jnp.float32),
            jax.ShapeDtypeStruct((e_pad,), jnp.int32),
            jax.ShapeDtypeStruct((nsc, 64), jnp.int32),
        ),
        mesh=mesh,
        scratch_types=[
            pltpu.VMEM((2 * epw,), jnp.int32),
            pltpu.VMEM((2 * epw,), jnp.int32),
            pltpu.VMEM((epw,), jnp.int32),
            pltpu.VMEM((epw,), jnp.int32),
            pltpu.VMEM((epw,), jnp.int32),
            pltpu.VMEM((NGRP * cap,), jnp.int32),
            pltpu.VMEM((NGRP * cap,), jnp.int32),
            pltpu.VMEM((NGRP * cap,), jnp.int32),
            pltpu.VMEM((tpsc, 16), jnp.int32),
            pltpu.VMEM((16,), jnp.int32),
            pltpu.VMEM((128,), jnp.int32),
            pltpu.VMEM((64,), jnp.int32),
            pltpu.VMEM((tslice,), jnp.int32),
            pltpu.VMEM((tslice,), jnp.int32),
            pltpu.VMEM((tslice,), jnp.int32),
            pltpu.VMEM((128, EMB), jnp.float32),
            pltpu.VMEM((128, EMB), jnp.float32),
            pltpu.VMEM_SHARED((REGION,), jnp.int32),
            pltpu.VMEM_SHARED((REGION,), jnp.int32),
            pltpu.VMEM_SHARED((REGION,), jnp.int32),
            pltpu.VMEM_SHARED((tpsc, 16), jnp.int32),
            pltpu.SemaphoreType.DMA,
            pltpu.SemaphoreType.DMA,
        ],
        compiler_params=pltpu.CompilerParams(
            use_tc_tiling_on_sc=False, needs_layout_passes=False),
    )
    def k(pairs_hbm, attr_hbm, table_hbm, h_hbm, t_hbm, r_hbm, grp_hbm,
          pairs_v, attr_v, hful, tful, rful, hbin, tbin, rbin,
          cnt_all, vtmp, negbuf, grpbuf, hs_v, ts_v, rs_v, hrow, trow,
          sh_h, sh_t, sh_r, sh_cnt, sem_h, sem_t):
        sc = lax.axis_index("c")
        tid = lax.axis_index("s")
        lanes = lax.iota(jnp.int32, 16)
        nnode = table_hbm.shape[0]

        pltpu.sync_copy(pairs_hbm.at[sc * tpsc + tid], pairs_v)
        pltpu.sync_copy(attr_hbm.at[sc * tpsc + tid], attr_v)

        def deint(j, carry):
            off = j * 32
            hful[pl.ds(j * 16, 16)] = plsc.load_gather(
                pairs_v, [lanes * 2 + off])
            tful[pl.ds(j * 16, 16)] = plsc.load_gather(
                pairs_v, [lanes * 2 + (off + 1)])
            rful[pl.ds(j * 16, 16)] = plsc.load_gather(
                attr_v, [lanes * 2 + (off + 1)])
            return carry

        lax.fori_loop(0, nvec, deint, 0)

        neg1 = jnp.full((16,), -1, jnp.int32)
        for g in range(NGRP):

            def rfill(j, carry, g=g):
                rbin[pl.ds(g * cap + j * 16, 16)] = neg1
                return carry

            lax.fori_loop(0, cap // 16, rfill, 0)
        for j in range(8):
            negbuf[pl.ds(j * 16, 16)] = neg1

        cnts = []
        for g in range(NGRP):

            def bing(j, off, g=g):
                sl = pl.ds(j * 16, 16)
                rv = rful[sl]
                m = lax.shift_right_logical(rv, 3) == g
                mi = m.astype(jnp.int32)
                pos = (g * cap + off) + plsc.cumsum(mi) - mi
                plsc.store_scatter(hbin, [pos], hful[sl], mask=m)
                plsc.store_scatter(tbin, [pos], tful[sl], mask=m)
                plsc.store_scatter(rbin, [pos], rv, mask=m)
                return off + jnp.sum(mi)

            cnts.append(lax.fori_loop(0, nvec, bing, jnp.int32(0)))

        claims = [((c + 127) // 128) * 128 for c in cnts]
        cvec = jnp.zeros((16,), jnp.int32)
        for g in range(NGRP):
            cvec = jnp.where(lanes == g, claims[g], cvec)
        vtmp[...] = cvec
        pltpu.sync_copy(vtmp, sh_cnt.at[tid])
        plsc.subcore_barrier()
        pltpu.sync_copy(sh_cnt, cnt_all)

        gstart = jnp.int32(0)
        base = []
        starts = []
        tots = []
        for g in range(NGRP):
            col = plsc.load_gather(cnt_all, [lanes, jnp.full((16,), g)])
            tot = jnp.sum(col)
            csum = plsc.cumsum(col)
            myexcl = jnp.sum(jnp.where(lanes == tid, csum - col, 0))
            starts.append(gstart)
            tots.append(tot)
            base.append(gstart + myexcl)
            gstart = gstart + ((tot + (BLK - 1)) // BLK) * BLK

        for g in range(NGRP):
            nchk = (cnts[g] + 127) // 128

            def copyg(j, carry, g=g, b=base[g]):
                src = pl.ds(g * cap + j * 128, 128)
                dst = pl.ds(pl.multiple_of(b + j * 128, 128), 128)
                pltpu.sync_copy(hbin.at[src], sh_h.at[dst])
                pltpu.sync_copy(tbin.at[src], sh_t.at[dst])
                pltpu.sync_copy(rbin.at[src], sh_r.at[dst])
                return carry

            lax.fori_loop(0, nchk, copyg, 0)

        @pl.when(tid == 0)
        def _():
            for g in range(NGRP):
                seg_end = starts[g] + ((tots[g] + (BLK - 1)) // BLK) * BLK
                ntail = (seg_end - (starts[g] + tots[g])) // 128

                def tfill(j, carry, s0=starts[g] + tots[g]):
                    dst = pl.ds(pl.multiple_of(s0 + j * 128, 128), 128)
                    pltpu.sync_copy(negbuf, sh_r.at[dst])
                    return carry

                lax.fori_loop(0, ntail, tfill, 0)

            nrt = (REGION - gstart) // 128

            def rtfill(j, carry, s0=gstart):
                dst = pl.ds(pl.multiple_of(s0 + j * 128, 128), 128)
                pltpu.sync_copy(negbuf, sh_r.at[dst])
                return carry

            lax.fori_loop(0, nrt, rtfill, 0)

            sb = [s // BLK for s in starts]
            for kb in range(4):
                bid = lanes + kb * 16
                gv = jnp.zeros((16,), jnp.int32)
                for g in range(1, NGRP):
                    gv = gv + jnp.where(bid >= sb[g], 1, 0).astype(jnp.int32)
                grpbuf[pl.ds(kb * 16, 16)] = gv
            pltpu.sync_copy(grpbuf, grp_hbm.at[sc])

        plsc.subcore_barrier()

        row0 = tid * tslice
        gbase = sc * REGION + row0
        pltpu.sync_copy(sh_h.at[pl.ds(row0, tslice)], hs_v)
        pltpu.sync_copy(sh_t.at[pl.ds(row0, tslice)], ts_v)
        pltpu.sync_copy(sh_r.at[pl.ds(row0, tslice)], rs_v)
        pltpu.sync_copy(rs_v, r_hbm.at[pl.ds(gbase, tslice)])

        def clamp(j, carry):
            sl = pl.ds(j * 16, 16)
            hs_v[sl] = jnp.clip(hs_v[sl], 0, nnode - 1)
            ts_v[sl] = jnp.clip(ts_v[sl], 0, nnode - 1)
            return carry

        lax.fori_loop(0, tslice // 16, clamp, 0)

        def gat(c, carry):
            cph = pltpu.async_copy(
                table_hbm.at[hs_v.at[pl.ds(c * 128, 128)]], hrow, sem_h)
            cpt = pltpu.async_copy(
                table_hbm.at[ts_v.at[pl.ds(c * 128, 128)]], trow, sem_t)
            cph.wait()
            cpt.wait()
            pltpu.sync_copy(hrow, h_hbm.at[pl.ds(gbase + c * 128, 128)])
            pltpu.sync_copy(trow, t_hbm.at[pl.ds(gbase + c * 128, 128)])
            return carry

        lax.fori_loop(0, n_ch, gat, 0)

    return k(pairs, attr, node_emb)


def _tc_loss_sum(grpflat, h, t, r_col, waug):
    e_pad = h.shape[0]
    nblk = e_pad // BLK
    gcols = RPG * REPR

    def body(grp_ref, h_ref, t_ref, r_ref, w_ref, out_ref):
        i = pl.program_id(0)
        g = grp_ref[(i // BPSC) * 64 + lax.rem(i, BPSC)]
        d = h_ref[...] - t_ref[...]
        q = r_ref[...] - 8 * g
        oh = (lax.broadcasted_iota(jnp.int32, (BLK, RPG), 1) == q)
        dp = jnp.concatenate(
            [d.astype(jnp.bfloat16), oh.astype(jnp.bfloat16)], axis=1)
        tt = jnp.dot(dp, w_ref[...], preferred_element_type=jnp.float32)
        colk = lax.shift_right_logical(
            lax.broadcasted_iota(jnp.int32, (BLK, gcols), 1), 5)
        sel = jnp.where(colk == q, tt, 0.0)
        s = jnp.sum(sel * sel)

        @pl.when(i == 0)
        def _():
            out_ref[...] = jnp.zeros_like(out_ref)

        out_ref[...] += s

    grid_spec = pltpu.PrefetchScalarGridSpec(
        num_scalar_prefetch=1,
        grid=(nblk,),
        in_specs=[
            pl.BlockSpec((BLK, EMB), lambda i, grp: (i, 0)),
            pl.BlockSpec((BLK, EMB), lambda i, grp: (i, 0)),
            pl.BlockSpec((BLK, 1), lambda i, grp: (i, 0)),
            pl.BlockSpec(
                (EMB + RPG, gcols),
                lambda i, grp: (0, grp[(i // BPSC) * 64 + lax.rem(i, BPSC)])),
        ],
        out_specs=pl.BlockSpec((1, 1), lambda i, grp: (0, 0)),
    )
    out = pl.pallas_call(
        body,
        grid_spec=grid_spec,
        out_shape=jax.ShapeDtypeStruct((1, 1), jnp.float32),
    )(grpflat, h, t, r_col, waug)
    return out[0, 0]


def kernel(edge_index_t, edge_attr, node_emb, r_emb_w, r_proj_w):
    e_total = edge_index_t.shape[0]
    nw = 32
    pairs = edge_index_t.reshape(nw, 2 * e_total // nw)
    attr = edge_attr.reshape(nw, 2 * e_total // nw)

    hs, ts, rs, grp = _sc_sort_gather(pairs, attr, node_emb)

    wt = r_proj_w.reshape(NREL, EMB, REPR).transpose(1, 0, 2).reshape(
        EMB, NREL * REPR)
    wb = jnp.broadcast_to(
        r_emb_w.reshape(NGRP, RPG, 1, REPR), (NGRP, RPG, RPG, REPR))
    wb = wb.transpose(1, 0, 2, 3).reshape(RPG, NREL * REPR)
    waug = jnp.concatenate([wt, wb], axis=0).astype(jnp.bfloat16)

    total = _tc_loss_sum(grp.reshape(-1), hs, ts, rs.reshape(-1, 1), waug)
    return total / jnp.float32(e_total * REPR)


# speedup vs baseline: 3.5771x; 3.5771x over previous
import functools

import jax
import jax.numpy as jnp
from jax import lax
from jax.experimental import pallas as pl
from jax.experimental.pallas import tpu as pltpu
from jax.experimental.pallas import tpu_sc as plsc

EMB = 64
REPR = 32
NREL = 64


def _sc_gather(idx2d, node_emb, n_rows):
    nw, n_ch, ch = idx2d.shape
    info = plsc.get_sparse_core_info()
    mesh = plsc.VectorSubcoreMesh(core_axis_name="c", subcore_axis_name="s")
    per_w = n_ch * ch

    @functools.partial(
        pl.kernel,
        out_type=jax.ShapeDtypeStruct((n_rows, EMB), jnp.float32),
        mesh=mesh,
        scratch_types=[
            pltpu.VMEM((n_ch, ch), jnp.int32),
            pltpu.VMEM((ch, EMB), jnp.float32),
            pltpu.SemaphoreType.DMA,
        ],
        compiler_params=pltpu.CompilerParams(use_tc_tiling_on_sc=False),
    )
    def k(idx_hbm, table_hbm, out_hbm, idx_v, rows_v, sem):
        wid = lax.axis_index("s") * info.num_cores + lax.axis_index("c")
        pltpu.sync_copy(idx_hbm.at[wid], idx_v)
        base = wid * per_w

        def body(c, carry):
            pltpu.async_copy(table_hbm.at[idx_v.at[c]], rows_v, sem).wait()
            pltpu.sync_copy(rows_v, out_hbm.at[pl.ds(base + c * ch, ch)])
            return carry

        lax.fori_loop(0, n_ch, body, 0)

    return k(idx2d, node_emb)


def _tc_loss_sum(x2, r_col, waug, block_e):
    e_total = x2.shape[0]
    nblk = e_total // block_e
    ncol = NREL * REPR

    def body(x_ref, r_ref, w_ref, out_ref):
        i = pl.program_id(0)
        x = x_ref[...]
        d = x[:, :EMB] - x[:, EMB:]
        r = r_ref[...]
        oh = (lax.broadcasted_iota(jnp.int32, (block_e, NREL), 1) == r)
        dp = jnp.concatenate(
            [d.astype(jnp.bfloat16), oh.astype(jnp.bfloat16)], axis=1)
        t = jnp.dot(dp, w_ref[...], preferred_element_type=jnp.float32)
        colrel = lax.shift_right_logical(
            lax.broadcasted_iota(jnp.int32, (block_e, ncol), 1), 5)
        sel = jnp.where(colrel == r, t, 0.0)
        s = jnp.sum(sel * sel)

        @pl.when(i == 0)
        def _():
            out_ref[...] = jnp.zeros_like(out_ref)

        out_ref[...] += s

    out = pl.pallas_call(
        body,
        grid=(nblk,),
        in_specs=[
            pl.BlockSpec((block_e, 2 * EMB), lambda i: (i, 0)),
            pl.BlockSpec((block_e, 1), lambda i: (i, 0)),
            pl.BlockSpec((2 * EMB, ncol), lambda i: (0, 0)),
        ],
        out_specs=pl.BlockSpec((1, 1), lambda i: (0, 0)),
        out_shape=jax.ShapeDtypeStruct((1, 1), jnp.float32),
    )(x2, r_col, waug)
    return out[0, 0]


def kernel(edge_index_t, edge_attr, node_emb, r_emb_w, r_proj_w):
    e_total = edge_index_t.shape[0]
    n_rows = 2 * e_total

    nw, ch = 32, 128
    n_ch = n_rows // (nw * ch)
    idx2d = edge_index_t.reshape(nw, n_ch, ch)

    x = _sc_gather(idx2d, node_emb, n_rows)
    x2 = x.reshape(e_total, 2 * EMB)

    wt = r_proj_w.reshape(NREL, EMB, REPR).transpose(1, 0, 2).reshape(
        EMB, NREL * REPR)
    wtile = jnp.broadcast_to(r_emb_w[:, None, :], (NREL, NREL, REPR)).reshape(
        NREL, NREL * REPR)
    waug = jnp.concatenate([wt, wtile], axis=0).astype(jnp.bfloat16)

    r_col = edge_attr[:, 1:2]

    total = _tc_loss_sum(x2, r_col, waug, block_e=512)
    return total / jnp.float32(e_total * REPR)


# speedup vs baseline: 3.6934x; 1.0325x over previous
import functools

import jax
import jax.numpy as jnp
from jax import lax
from jax.experimental import pallas as pl
from jax.experimental.pallas import tpu as pltpu
from jax.experimental.pallas import tpu_sc as plsc

EMB = 64
REPR = 32
NREL = 64


def _sc_gather(idx2d, node_emb, n_rows):
    nw, n_ch, ch = idx2d.shape
    info = plsc.get_sparse_core_info()
    mesh = plsc.VectorSubcoreMesh(core_axis_name="c", subcore_axis_name="s")
    per_w = n_ch * ch

    @functools.partial(
        pl.kernel,
        out_type=jax.ShapeDtypeStruct((n_rows, EMB), jnp.float32),
        mesh=mesh,
        scratch_types=[
            pltpu.VMEM((n_ch, ch), jnp.int32),
            pltpu.VMEM((ch, EMB), jnp.float32),
            pltpu.VMEM((ch, EMB), jnp.float32),
            pltpu.SemaphoreType.DMA,
            pltpu.SemaphoreType.DMA,
        ],
        compiler_params=pltpu.CompilerParams(use_tc_tiling_on_sc=False),
    )
    def k(idx_hbm, table_hbm, out_hbm, idx_v, rows0, rows1, sem0, sem1):
        wid = lax.axis_index("s") * info.num_cores + lax.axis_index("c")
        pltpu.sync_copy(idx_hbm.at[wid], idx_v)
        base = wid * per_w

        pltpu.async_copy(table_hbm.at[idx_v.at[0]], rows0, sem0)

        def body(c2, carry):
            c = c2 * 2
            pltpu.async_copy(table_hbm.at[idx_v.at[c + 1]], rows1, sem1)
            pltpu.make_async_copy(
                table_hbm.at[idx_v.at[c]], rows0, sem0).wait()
            pltpu.sync_copy(rows0, out_hbm.at[pl.ds(base + c * ch, ch)])

            @pl.when(c2 + 1 < n_ch // 2)
            def _():
                pltpu.async_copy(table_hbm.at[idx_v.at[c + 2]], rows0, sem0)

            pltpu.make_async_copy(
                table_hbm.at[idx_v.at[c + 1]], rows1, sem1).wait()
            pltpu.sync_copy(rows1, out_hbm.at[pl.ds(base + (c + 1) * ch, ch)])
            return carry

        lax.fori_loop(0, n_ch // 2, body, 0)

    return k(idx2d, node_emb)


def _tc_loss_sum(x2, r_col, waug, block_e):
    e_total = x2.shape[0]
    nblk = e_total // block_e
    ncol = NREL * REPR

    def body(x_ref, r_ref, w_ref, out_ref):
        i = pl.program_id(0)
        x = x_ref[...]
        d = x[:, :EMB] - x[:, EMB:]
        r = r_ref[...]
        oh = (lax.broadcasted_iota(jnp.int32, (block_e, NREL), 1) == r)
        dp = jnp.concatenate(
            [d.astype(jnp.bfloat16), oh.astype(jnp.bfloat16)], axis=1)
        t = jnp.dot(dp, w_ref[...], preferred_element_type=jnp.float32)
        colrel = lax.shift_right_logical(
            lax.broadcasted_iota(jnp.int32, (block_e, ncol), 1), 5)
        sel = jnp.where(colrel == r, t, 0.0)
        s = jnp.sum(sel * sel)

        @pl.when(i == 0)
        def _():
            out_ref[...] = jnp.zeros_like(out_ref)

        out_ref[...] += s

    out = pl.pallas_call(
        body,
        grid=(nblk,),
        in_specs=[
            pl.BlockSpec((block_e, 2 * EMB), lambda i: (i, 0)),
            pl.BlockSpec((block_e, 1), lambda i: (i, 0)),
            pl.BlockSpec((2 * EMB, ncol), lambda i: (0, 0)),
        ],
        out_specs=pl.BlockSpec((1, 1), lambda i: (0, 0)),
        out_shape=jax.ShapeDtypeStruct((1, 1), jnp.float32),
    )(x2, r_col, waug)
    return out[0, 0]


def kernel(edge_index_t, edge_attr, node_emb, r_emb_w, r_proj_w):
    e_total = edge_index_t.shape[0]
    n_rows = 2 * e_total

    nw, ch = 32, 128
    n_ch = n_rows // (nw * ch)
    idx2d = edge_index_t.reshape(nw, n_ch, ch)

    x = _sc_gather(idx2d, node_emb, n_rows)
    x2 = x.reshape(e_total, 2 * EMB)

    wt = r_proj_w.reshape(NREL, EMB, REPR).transpose(1, 0, 2).reshape(
        EMB, NREL * REPR)
    wtile = jnp.broadcast_to(r_emb_w[:, None, :], (NREL, NREL, REPR)).reshape(
        NREL, NREL * REPR)
    waug = jnp.concatenate([wt, wtile], axis=0).astype(jnp.bfloat16)

    r_col = edge_attr[:, 1:2]

    total = _tc_loss_sum(x2, r_col, waug, block_e=512)
    return total / jnp.float32(e_total * REPR)
